# baseline (speedup 1.0000x reference)
import functools

import jax
import jax.numpy as jnp
from jax import lax
from jax.experimental import pallas as pl
from jax.experimental.pallas import tpu as pltpu
from jax.experimental.pallas import tpu_sc as plsc

_LANES = 128


def _gather_rows_sc(table_t, table_tail_t, idx):
    embed, n_full = table_t.shape
    n_main = ((n_full - 1) // _LANES) * _LANES
    w_tail = table_tail_t.shape[1]
    batch = idx.shape[0]
    info = plsc.get_sparse_core_info()
    nc, ns, nl = info.num_cores, info.num_subcores, info.num_lanes
    nw = nc * ns
    b_per_w = batch // nw
    mesh = plsc.VectorSubcoreMesh(core_axis_name="c", subcore_axis_name="s")

    @functools.partial(
        pl.kernel,
        mesh=mesh,
        compiler_params=pltpu.CompilerParams(needs_layout_passes=False),
        out_type=jax.ShapeDtypeStruct((batch, embed), jnp.float32),
        scratch_types=[
            pltpu.VMEM((b_per_w,), jnp.int32),
            pltpu.VMEM((8, embed, _LANES), jnp.float32),
            pltpu.VMEM((embed, w_tail), jnp.float32),
            pltpu.VMEM((b_per_w, embed), jnp.float32),
            pltpu.SemaphoreType.DMA((8,)),
            pltpu.SemaphoreType.DMA,
        ],
    )
    def gather_kernel(table_hbm, tail_hbm, idx_hbm, out_hbm, idx_v, slab_v,
                      tail_v, rows_v, sems, osem):
        nbuf = 8
        wid = lax.axis_index("s") * nc + lax.axis_index("c")
        base = wid * b_per_w
        pltpu.sync_copy(idx_hbm.at[pl.ds(base, b_per_w)], idx_v)
        pltpu.sync_copy(tail_hbm, tail_v)
        lanes, starts, tails = [], [], []
        for g in range(b_per_w // nl):
            vec = idx_v[pl.ds(g * nl, nl)]
            for j in range(nl):
                i = vec[j]
                in_tail = i >= n_main
                lane = lax.rem(i, _LANES)
                start = pl.multiple_of(
                    jnp.where(in_tail, 0, i - lane), _LANES)
                lanes.append(lane)
                starts.append(start)
                tails.append(in_tail)

        def fetch(b):
            return pltpu.async_copy(
                table_hbm.at[:, pl.ds(starts[b], _LANES)],
                slab_v.at[b % nbuf], sems.at[b % nbuf])

        pend = [fetch(b) for b in range(nbuf)]
        row_ids = lax.iota(jnp.int32, nl)
        for b in range(b_per_w):
            pend[b % nbuf].wait()
            col = jnp.full((nl,), lanes[b], jnp.int32)
            col_t = jnp.full((nl,), jnp.minimum(lanes[b], w_tail - 1),
                             jnp.int32)
            sel = jnp.full((nl,), tails[b], jnp.bool_)
            for g in range(embed // nl):
                vals = plsc.load_gather(
                    slab_v.at[b % nbuf], [row_ids + g * nl, col])
                tvals = plsc.load_gather(tail_v, [row_ids + g * nl, col_t])
                rows_v[b, pl.ds(g * nl, nl)] = jnp.where(sel, tvals, vals)
            if b + nbuf < b_per_w:
                pend[b % nbuf] = fetch(b + nbuf)
        pltpu.async_copy(rows_v, out_hbm.at[pl.ds(base, b_per_w)], osem).wait()

    return gather_kernel(table_t, table_tail_t, idx)


_BLOCK_N = 2048


def _score_t_tc(users_emb, items_t):
    batch, embed = users_emb.shape
    n = items_t.shape[1]

    def body(u_ref, it_ref, o_ref):
        s = lax.dot_general(
            it_ref[...], u_ref[...], (((0,), (1,)), ((), ())),
            preferred_element_type=jnp.float32,
        )
        o_ref[...] = 0.5 * jnp.tanh(0.5 * s) + 0.5

    return pl.pallas_call(
        body,
        grid=(pl.cdiv(n, _BLOCK_N),),
        in_specs=[
            pl.BlockSpec((batch, embed), lambda j: (0, 0)),
            pl.BlockSpec((embed, _BLOCK_N), lambda j: (0, j)),
        ],
        out_specs=pl.BlockSpec((_BLOCK_N, batch), lambda j: (j, 0)),
        out_shape=jax.ShapeDtypeStruct((n, batch), jnp.float32),
    )(users_emb, items_t)


def kernel(users, embedding_user_weight, embedding_item_weight):
    idx = users.astype(jnp.int32)
    n_users = embedding_user_weight.shape[0]
    n_main = ((n_users - 1) // _LANES) * _LANES
    tail_t = embedding_user_weight[n_main:].T
    users_emb = _gather_rows_sc(embedding_user_weight.T, tail_t, idx)
    out_t = _score_t_tc(users_emb, embedding_item_weight.T)
    return out_t.T
